# baseline (speedup 1.0000x reference)
import jax
import jax.numpy as jnp
from jax import lax
from jax.experimental import pallas as pl
from jax.experimental.pallas import tpu as pltpu

RANK = 64
BLK = 512
NP_PAD = 1024


def _issue(tab_ref, idx_ref, base, buf, slot, sem):
    def one(j, _):
        idx = idx_ref[base + j]
        pltpu.make_async_copy(
            tab_ref.at[pl.ds(lax.shift_right_logical(idx, 1), 1)],
            buf.at[slot, pl.ds(j, 1)],
            sem.at[slot],
        ).start()
        return 0

    lax.fori_loop(0, BLK, one, 0, unroll=8)


def _wait(buf, slot, sem):
    pltpu.make_async_copy(buf.at[slot], buf.at[slot], sem.at[slot]).wait()


def _half(pair, idx3_ref):
    odd = (idx3_ref[0] & 1) == 1
    return jnp.where(odd, pair[:, RANK:], pair[:, :RANK]).astype(jnp.bfloat16)


def _body(si_ref, oi_ref, tab_ref, si3_ref, oi3_ref, p_ref, pt_ref,
          sm_ref, tm_ref, out_ref, sbuf, obuf, sem_s, sem_o):
    i = pl.program_id(0)
    n = pl.num_programs(0)
    slot = lax.rem(i, 2)

    @pl.when(i == 0)
    def _prologue():
        _issue(tab_ref, si_ref, 0, sbuf, 0, sem_s)
        _issue(tab_ref, oi_ref, 0, obuf, 0, sem_o)

    @pl.when(i + 1 < n)
    def _prefetch_next():
        nxt = lax.rem(i + 1, 2)
        _issue(tab_ref, si_ref, (i + 1) * BLK, sbuf, nxt, sem_s)
        _issue(tab_ref, oi_ref, (i + 1) * BLK, obuf, nxt, sem_o)

    _wait(sbuf, slot, sem_s)
    _wait(obuf, slot, sem_o)

    s = _half(sbuf[slot], si3_ref)
    o = _half(obuf[slot], oi3_ref)
    s_rep = lax.dot_general(s, sm_ref[...], (((1,), (0,)), ((), ())),
                            preferred_element_type=jnp.float32)
    o_tile = lax.dot_general(o, tm_ref[...], (((1,), (0,)), ((), ())),
                             preferred_element_type=jnp.float32)
    x = (s_rep * o_tile).astype(jnp.bfloat16)
    scores = lax.dot_general(x, pt_ref[...], (((1,), (0,)), ((), ())),
                             preferred_element_type=jnp.float32)
    pidx = p_ref[0]
    sel = pidx == lax.broadcasted_iota(jnp.int32, (BLK, NP_PAD), 1)
    spo = jnp.sum(jnp.where(sel, scores, 0.0), axis=1, keepdims=True)
    out_ref[...] = jax.nn.sigmoid(spo)


def kernel(s_input, p_input, o_input, entity_table, predicate_table):
    b = s_input.shape[0]
    ne = entity_table.shape[0]
    np_real = predicate_table.shape[0]
    s_idx = s_input.reshape(b).astype(jnp.int32)
    o_idx = o_input.reshape(b).astype(jnp.int32)
    si3 = s_input.reshape(b // BLK, BLK, 1).astype(jnp.int32)
    oi3 = o_input.reshape(b // BLK, BLK, 1).astype(jnp.int32)
    p3 = p_input.reshape(b // BLK, BLK, 1).astype(jnp.int32)

    tab2 = jnp.concatenate(
        [entity_table[0::2, :], entity_table[1::2, :]], axis=1)

    ptt = predicate_table.reshape(np_real, RANK * RANK).astype(jnp.bfloat16)
    ptt = jnp.pad(ptt, ((0, NP_PAD - np_real), (0, 0))).T

    k = jnp.arange(RANK * RANK, dtype=jnp.int32)
    ar = jnp.arange(RANK, dtype=jnp.int32)
    sm = (ar[:, None] == k[None, :] // RANK).astype(jnp.bfloat16)
    tm = (ar[:, None] == k[None, :] % RANK).astype(jnp.bfloat16)

    grid_spec = pltpu.PrefetchScalarGridSpec(
        num_scalar_prefetch=2,
        grid=(b // BLK,),
        in_specs=[
            pl.BlockSpec(memory_space=pl.ANY),
            pl.BlockSpec((1, BLK, 1), lambda i, si, oi: (i, 0, 0)),
            pl.BlockSpec((1, BLK, 1), lambda i, si, oi: (i, 0, 0)),
            pl.BlockSpec((1, BLK, 1), lambda i, si, oi: (i, 0, 0)),
            pl.BlockSpec((RANK * RANK, NP_PAD), lambda i, si, oi: (0, 0)),
            pl.BlockSpec((RANK, RANK * RANK), lambda i, si, oi: (0, 0)),
            pl.BlockSpec((RANK, RANK * RANK), lambda i, si, oi: (0, 0)),
        ],
        out_specs=pl.BlockSpec((BLK, 1), lambda i, si, oi: (i, 0)),
        scratch_shapes=[
            pltpu.VMEM((2, BLK, 2 * RANK), jnp.float32),
            pltpu.VMEM((2, BLK, 2 * RANK), jnp.float32),
            pltpu.SemaphoreType.DMA((2,)),
            pltpu.SemaphoreType.DMA((2,)),
        ],
    )
    out = pl.pallas_call(
        _body,
        grid_spec=grid_spec,
        out_shape=jax.ShapeDtypeStruct((b, 1), jnp.float32),
    )(s_idx, o_idx, tab2, si3, oi3, p3, ptt, sm, tm)
    return out


# speedup vs baseline: 13.5779x; 13.5779x over previous
import jax
import jax.numpy as jnp
from jax import lax
from jax.experimental import pallas as pl
from jax.experimental.pallas import tpu as pltpu

RANK = 64
BLK = 512
NP_PAD = 1024


def _issue(tab_ref, si_ref, oi_ref, base, sbuf, obuf, slot, sem_s, sem_o):
    def one(j, _):
        si = si_ref[base + j]
        oi = oi_ref[base + j]
        pltpu.make_async_copy(
            tab_ref.at[pl.ds(si, 1)],
            sbuf.at[slot, pl.ds(j, 1)],
            sem_s.at[slot],
        ).start()
        pltpu.make_async_copy(
            tab_ref.at[pl.ds(oi, 1)],
            obuf.at[slot, pl.ds(j, 1)],
            sem_o.at[slot],
        ).start()
        return 0

    lax.fori_loop(0, BLK, one, 0, unroll=8)


def _wait(buf, slot, sem):
    pltpu.make_async_copy(buf.at[slot], buf.at[slot], sem.at[slot]).wait()


def _body(si_ref, oi_ref, tab_ref, p_ref, pt_ref, sm_ref, tm_ref, out_ref,
          sbuf, obuf, sem_s, sem_o):
    i = pl.program_id(0)
    n = pl.num_programs(0)
    slot = lax.rem(i, 2)

    @pl.when(i == 0)
    def _prologue():
        _issue(tab_ref, si_ref, oi_ref, 0, sbuf, obuf, 0, sem_s, sem_o)

    @pl.when(i + 1 < n)
    def _prefetch_next():
        nxt = lax.rem(i + 1, 2)
        _issue(tab_ref, si_ref, oi_ref, (i + 1) * BLK, sbuf, obuf, nxt,
               sem_s, sem_o)

    _wait(sbuf, slot, sem_s)
    _wait(obuf, slot, sem_o)

    s = sbuf[slot].astype(jnp.bfloat16)
    o = obuf[slot].astype(jnp.bfloat16)
    s_rep = lax.dot_general(s, sm_ref[...], (((1,), (0,)), ((), ())),
                            preferred_element_type=jnp.float32)
    o_tile = lax.dot_general(o, tm_ref[...], (((1,), (0,)), ((), ())),
                             preferred_element_type=jnp.float32)
    x = (s_rep * o_tile).astype(jnp.bfloat16)
    scores = lax.dot_general(x, pt_ref[...], (((1,), (0,)), ((), ())),
                             preferred_element_type=jnp.float32)
    pidx = p_ref[0]
    sel = pidx == lax.broadcasted_iota(jnp.int32, (BLK, NP_PAD), 1)
    spo = jnp.sum(jnp.where(sel, scores, 0.0), axis=1, keepdims=True)
    out_ref[...] = jax.nn.sigmoid(spo)


def kernel(s_input, p_input, o_input, entity_table, predicate_table):
    b = s_input.shape[0]
    np_real = predicate_table.shape[0]
    s_idx = s_input.reshape(b).astype(jnp.int32)
    o_idx = o_input.reshape(b).astype(jnp.int32)
    p3 = p_input.reshape(b // BLK, BLK, 1).astype(jnp.int32)

    ptt = predicate_table.reshape(np_real, RANK * RANK).astype(jnp.bfloat16)
    ptt = jnp.pad(ptt, ((0, NP_PAD - np_real), (0, 0))).T

    k = jnp.arange(RANK * RANK, dtype=jnp.int32)
    ar = jnp.arange(RANK, dtype=jnp.int32)
    sm = (ar[:, None] == k[None, :] // RANK).astype(jnp.bfloat16)
    tm = (ar[:, None] == k[None, :] % RANK).astype(jnp.bfloat16)

    grid_spec = pltpu.PrefetchScalarGridSpec(
        num_scalar_prefetch=2,
        grid=(b // BLK,),
        in_specs=[
            pl.BlockSpec(memory_space=pl.ANY),
            pl.BlockSpec((1, BLK, 1), lambda i, si, oi: (i, 0, 0)),
            pl.BlockSpec((RANK * RANK, NP_PAD), lambda i, si, oi: (0, 0)),
            pl.BlockSpec((RANK, RANK * RANK), lambda i, si, oi: (0, 0)),
            pl.BlockSpec((RANK, RANK * RANK), lambda i, si, oi: (0, 0)),
        ],
        out_specs=pl.BlockSpec((BLK, 1), lambda i, si, oi: (i, 0)),
        scratch_shapes=[
            pltpu.VMEM((2, BLK, RANK), jnp.float32),
            pltpu.VMEM((2, BLK, RANK), jnp.float32),
            pltpu.SemaphoreType.DMA((2,)),
            pltpu.SemaphoreType.DMA((2,)),
        ],
    )
    out = pl.pallas_call(
        _body,
        grid_spec=grid_spec,
        out_shape=jax.ShapeDtypeStruct((b, 1), jnp.float32),
    )(s_idx, o_idx, entity_table, p3, ptt, sm, tm)
    return out
